# baseline (speedup 1.0000x reference)
import functools

import jax
import jax.numpy as jnp
from jax import lax
from jax.experimental import pallas as pl
from jax.experimental.pallas import tpu as pltpu
from jax.experimental.pallas import tpu_sc as plsc

B = 1024
D_IN = 512
LATENT = 64
SOM_R = 32
SOM_C = 32
NCODE = SOM_R * SOM_C
BLK = 256
GRID = B // BLK
NSLOT = 5

_TROWS = NCODE + 8
_TW = 128
_BIG = (1 << 30)


def _tc_body(x_ref, we_ref, be_ref, wq_ref, bq_ref, wde_ref, bde_ref,
             e_ref, et_ref,
             xe_ref, xq_ref, ze_ref, zq_ref, zd_ref, k_ref, g_ref, tab_ref):
    x = x_ref[...]
    E = e_ref[...]
    Et = et_ref[...]
    ze = jnp.dot(x, we_ref[...], preferred_element_type=jnp.float32) + be_ref[...]
    ze_ref[...] = ze

    enorm = jnp.sum(Et * Et, axis=0, keepdims=True)
    cross = jnp.dot(ze, Et, preferred_element_type=jnp.float32,
                    precision=lax.Precision.HIGHEST)
    c = enorm - 2.0 * cross
    znorm = jnp.sum(ze * ze, axis=1, keepdims=True)
    zd_ref[...] = znorm + c

    cmin = jnp.min(c, axis=1, keepdims=True)
    iota = lax.broadcasted_iota(jnp.int32, (BLK, NCODE), 1)
    k = jnp.min(jnp.where(c <= cmin, iota, _BIG), axis=1, keepdims=True)
    k_ref[...] = k

    onehot = (iota == k).astype(jnp.float32)
    zq = jnp.dot(onehot, E, preferred_element_type=jnp.float32,
                 precision=lax.Precision.HIGHEST)
    zq_ref[...] = zq
    xq_ref[...] = jnp.dot(zq, wq_ref[...], preferred_element_type=jnp.float32) + bq_ref[...]
    xe_ref[...] = jnp.dot(ze, wde_ref[...], preferred_element_type=jnp.float32) + bde_ref[...]

    k1 = k // SOM_C
    k2 = k % SOM_C
    up = jnp.where(k1 < (SOM_R - 1), k + SOM_C, NCODE)
    down = jnp.where(k1 > 0, k - SOM_C, NCODE)
    left = jnp.where(k2 > 0, k - 1, NCODE)
    zero = jnp.full((BLK, 1), NCODE, jnp.int32)
    g_ref[...] = jnp.concatenate([k, up, down, zero, left], axis=1)

    @pl.when(pl.program_id(0) == 0)
    def _():
        tab_ref[...] = jnp.zeros((_TROWS, _TW), jnp.float32)
        tab_ref[pl.ds(0, NCODE), pl.ds(0, LATENT)] = E


def _tc_call(x, W_enc, b_enc2, W_dec_q, b_dec_q2, W_dec_e, b_dec_e2,
             E_flat, E_t):
    full = lambda s: pl.BlockSpec(s, lambda i: (0,) * len(s))
    return pl.pallas_call(
        _tc_body,
        grid=(GRID,),
        in_specs=[
            pl.BlockSpec((BLK, D_IN), lambda i: (i, 0)),
            full((D_IN, LATENT)),
            full((1, LATENT)),
            full((LATENT, D_IN)),
            full((1, D_IN)),
            full((LATENT, D_IN)),
            full((1, D_IN)),
            full((NCODE, LATENT)),
            full((LATENT, NCODE)),
        ],
        out_specs=[
            pl.BlockSpec((BLK, D_IN), lambda i: (i, 0)),
            pl.BlockSpec((BLK, D_IN), lambda i: (i, 0)),
            pl.BlockSpec((BLK, LATENT), lambda i: (i, 0)),
            pl.BlockSpec((BLK, LATENT), lambda i: (i, 0)),
            pl.BlockSpec((BLK, NCODE), lambda i: (i, 0)),
            pl.BlockSpec((BLK, 1), lambda i: (i, 0)),
            pl.BlockSpec((BLK, NSLOT), lambda i: (i, 0)),
            full((_TROWS, _TW)),
        ],
        out_shape=[
            jax.ShapeDtypeStruct((B, D_IN), jnp.float32),
            jax.ShapeDtypeStruct((B, D_IN), jnp.float32),
            jax.ShapeDtypeStruct((B, LATENT), jnp.float32),
            jax.ShapeDtypeStruct((B, LATENT), jnp.float32),
            jax.ShapeDtypeStruct((B, NCODE), jnp.float32),
            jax.ShapeDtypeStruct((B, 1), jnp.int32),
            jax.ShapeDtypeStruct((B, NSLOT), jnp.int32),
            jax.ShapeDtypeStruct((_TROWS, _TW), jnp.float32),
        ],
        compiler_params=pltpu.CompilerParams(
            dimension_semantics=("arbitrary",),
        ),
    )(x, W_enc, b_enc2, W_dec_q, b_dec_q2, W_dec_e, b_dec_e2, E_flat, E_t)


_NC = 2
_NS = 16
_NW = _NC * _NS
_NG = NSLOT * B
_BPW = _NG // _NW
_CH = _BPW // 2


@functools.lru_cache(maxsize=None)
def _sc_gather_fn():
    mesh = plsc.VectorSubcoreMesh(core_axis_name="c", subcore_axis_name="s")

    @functools.partial(
        pl.kernel,
        mesh=mesh,
        out_type=jax.ShapeDtypeStruct((_NG, _TW), jnp.float32),
        scratch_types=[
            pltpu.VMEM((_CH,), jnp.int32),
            pltpu.VMEM((_CH,), jnp.int32),
            pltpu.VMEM((_CH, _TW), jnp.float32),
            pltpu.VMEM((_CH, _TW), jnp.float32),
            pltpu.SemaphoreType.DMA,
            pltpu.SemaphoreType.DMA,
        ],
    )
    def _sc_gather(table_hbm, idx_hbm, out_hbm,
                   idx_a, idx_b, rows_a, rows_b, sem_a, sem_b):
        wid = lax.axis_index("s") * _NC + lax.axis_index("c")
        base = wid * _BPW
        pltpu.sync_copy(idx_hbm.at[pl.ds(base, _CH)], idx_a)
        pltpu.sync_copy(idx_hbm.at[pl.ds(base + _CH, _CH)], idx_b)
        ca = pltpu.async_copy(table_hbm.at[idx_a], rows_a, sem_a)
        cb = pltpu.async_copy(table_hbm.at[idx_b], rows_b, sem_b)
        ca.wait()
        pltpu.sync_copy(rows_a, out_hbm.at[pl.ds(base, _CH)])
        cb.wait()
        pltpu.sync_copy(rows_b, out_hbm.at[pl.ds(base + _CH, _CH)])

    return _sc_gather


def kernel(x, W_enc, b_enc, W_dec_q, b_dec_q, W_dec_e, b_dec_e, embeddings):
    E_flat = embeddings.reshape(NCODE, LATENT)
    x_e, x_q, z_e, z_q, zdist, k2d, gidx, table = _tc_call(
        x, W_enc, b_enc.reshape(1, LATENT),
        W_dec_q, b_dec_q.reshape(1, D_IN),
        W_dec_e, b_dec_e.reshape(1, D_IN),
        E_flat, E_flat.T)
    k = k2d.reshape(B)

    nb = _sc_gather_fn()(table, gidx.reshape(_NG))
    z_q_neighbors = nb[:, :LATENT].reshape(B, NSLOT, LATENT)
    return (x_e, x_q, z_e, z_q, z_q_neighbors, k, zdist)


# speedup vs baseline: 1.5502x; 1.5502x over previous
import functools

import jax
import jax.numpy as jnp
from jax import lax
from jax.experimental import pallas as pl
from jax.experimental.pallas import tpu as pltpu
from jax.experimental.pallas import tpu_sc as plsc

B = 1024
D_IN = 512
LATENT = 64
SOM_R = 32
SOM_C = 32
NCODE = SOM_R * SOM_C
BLK = 256
GRID = B // BLK
NSLOT = 5

_TROWS = NCODE + 8
_TW = 128
_BIG = (1 << 30)


def _tc_body(x_ref, we_ref, be_ref, wq_ref, bq_ref, wde_ref, bde_ref,
             e_ref, et_ref,
             xe_ref, xq_ref, ze_ref, zq_ref, zd_ref, k_ref, g_ref, tab_ref):
    x = x_ref[...]
    E = e_ref[...]
    Et = et_ref[...]
    ze = jnp.dot(x, we_ref[...], preferred_element_type=jnp.float32) + be_ref[...]
    ze_ref[...] = ze

    enorm = jnp.sum(Et * Et, axis=0, keepdims=True)
    cross = jnp.dot(ze, Et, preferred_element_type=jnp.float32,
                    precision=lax.Precision.HIGHEST)
    c = enorm - 2.0 * cross
    znorm = jnp.sum(ze * ze, axis=1, keepdims=True)
    zd_ref[...] = znorm + c

    cmin = jnp.min(c, axis=1, keepdims=True)
    iota = lax.broadcasted_iota(jnp.int32, (BLK, NCODE), 1)
    k = jnp.min(jnp.where(c <= cmin, iota, _BIG), axis=1, keepdims=True)
    k_ref[...] = k

    onehot = (iota == k).astype(jnp.float32)
    zq = jnp.dot(onehot, E, preferred_element_type=jnp.float32,
                 precision=lax.Precision.HIGHEST)
    zq_ref[...] = zq
    xq_ref[...] = jnp.dot(zq, wq_ref[...], preferred_element_type=jnp.float32) + bq_ref[...]
    xe_ref[...] = jnp.dot(ze, wde_ref[...], preferred_element_type=jnp.float32) + bde_ref[...]

    k1 = k // SOM_C
    k2 = k % SOM_C
    up = jnp.where(k1 < (SOM_R - 1), k + SOM_C, NCODE)
    down = jnp.where(k1 > 0, k - SOM_C, NCODE)
    left = jnp.where(k2 > 0, k - 1, NCODE)
    row_iota = lax.broadcasted_iota(jnp.int32, (BLK, 1), 0)
    zero = NCODE + (row_iota & 7)
    g_ref[...] = jnp.concatenate([k, up, down, zero, left], axis=1)

    @pl.when(pl.program_id(0) == 0)
    def _():
        tab_ref[...] = jnp.zeros((_TROWS, _TW), jnp.float32)
        tab_ref[pl.ds(0, NCODE), pl.ds(0, LATENT)] = E


def _tc_call(x, W_enc, b_enc2, W_dec_q, b_dec_q2, W_dec_e, b_dec_e2,
             E_flat, E_t):
    full = lambda s: pl.BlockSpec(s, lambda i: (0,) * len(s))
    return pl.pallas_call(
        _tc_body,
        grid=(GRID,),
        in_specs=[
            pl.BlockSpec((BLK, D_IN), lambda i: (i, 0)),
            full((D_IN, LATENT)),
            full((1, LATENT)),
            full((LATENT, D_IN)),
            full((1, D_IN)),
            full((LATENT, D_IN)),
            full((1, D_IN)),
            full((NCODE, LATENT)),
            full((LATENT, NCODE)),
        ],
        out_specs=[
            pl.BlockSpec((BLK, D_IN), lambda i: (i, 0)),
            pl.BlockSpec((BLK, D_IN), lambda i: (i, 0)),
            pl.BlockSpec((BLK, LATENT), lambda i: (i, 0)),
            pl.BlockSpec((BLK, LATENT), lambda i: (i, 0)),
            pl.BlockSpec((BLK, NCODE), lambda i: (i, 0)),
            pl.BlockSpec((BLK, 1), lambda i: (i, 0)),
            pl.BlockSpec((BLK, NSLOT), lambda i: (i, 0)),
            full((_TROWS, _TW)),
        ],
        out_shape=[
            jax.ShapeDtypeStruct((B, D_IN), jnp.float32),
            jax.ShapeDtypeStruct((B, D_IN), jnp.float32),
            jax.ShapeDtypeStruct((B, LATENT), jnp.float32),
            jax.ShapeDtypeStruct((B, LATENT), jnp.float32),
            jax.ShapeDtypeStruct((B, NCODE), jnp.float32),
            jax.ShapeDtypeStruct((B, 1), jnp.int32),
            jax.ShapeDtypeStruct((B, NSLOT), jnp.int32),
            jax.ShapeDtypeStruct((_TROWS, _TW), jnp.float32),
        ],
        compiler_params=pltpu.CompilerParams(
            dimension_semantics=("arbitrary",),
        ),
    )(x, W_enc, b_enc2, W_dec_q, b_dec_q2, W_dec_e, b_dec_e2, E_flat, E_t)


_NC = 2
_NS = 16
_NW = _NC * _NS
_NG = NSLOT * B
_BPW = _NG // _NW
_CH = _BPW // 2


@functools.lru_cache(maxsize=None)
def _sc_gather_fn():
    mesh = plsc.VectorSubcoreMesh(core_axis_name="c", subcore_axis_name="s")

    @functools.partial(
        pl.kernel,
        mesh=mesh,
        out_type=jax.ShapeDtypeStruct((_NG, _TW), jnp.float32),
        scratch_types=[
            pltpu.VMEM((_CH,), jnp.int32),
            pltpu.VMEM((_CH,), jnp.int32),
            pltpu.VMEM((_CH, _TW), jnp.float32),
            pltpu.VMEM((_CH, _TW), jnp.float32),
            pltpu.SemaphoreType.DMA,
            pltpu.SemaphoreType.DMA,
        ],
    )
    def _sc_gather(table_hbm, idx_hbm, out_hbm,
                   idx_a, idx_b, rows_a, rows_b, sem_a, sem_b):
        wid = lax.axis_index("s") * _NC + lax.axis_index("c")
        base = wid * _BPW
        pltpu.sync_copy(idx_hbm.at[pl.ds(base, _CH)], idx_a)
        pltpu.sync_copy(idx_hbm.at[pl.ds(base + _CH, _CH)], idx_b)
        ca = pltpu.async_copy(table_hbm.at[idx_a], rows_a, sem_a)
        cb = pltpu.async_copy(table_hbm.at[idx_b], rows_b, sem_b)
        ca.wait()
        pltpu.sync_copy(rows_a, out_hbm.at[pl.ds(base, _CH)])
        cb.wait()
        pltpu.sync_copy(rows_b, out_hbm.at[pl.ds(base + _CH, _CH)])

    return _sc_gather


def kernel(x, W_enc, b_enc, W_dec_q, b_dec_q, W_dec_e, b_dec_e, embeddings):
    E_flat = embeddings.reshape(NCODE, LATENT)
    x_e, x_q, z_e, z_q, zdist, k2d, gidx, table = _tc_call(
        x, W_enc, b_enc.reshape(1, LATENT),
        W_dec_q, b_dec_q.reshape(1, D_IN),
        W_dec_e, b_dec_e.reshape(1, D_IN),
        E_flat, E_flat.T)
    k = k2d.reshape(B)

    nb = _sc_gather_fn()(table, gidx.reshape(_NG))
    z_q_neighbors = nb[:, :LATENT].reshape(B, NSLOT, LATENT)
    return (x_e, x_q, z_e, z_q, z_q_neighbors, k, zdist)


# speedup vs baseline: 1.8904x; 1.2195x over previous
import functools

import jax
import jax.numpy as jnp
from jax import lax
from jax.experimental import pallas as pl
from jax.experimental.pallas import tpu as pltpu
from jax.experimental.pallas import tpu_sc as plsc

B = 1024
D_IN = 512
LATENT = 64
SOM_R = 32
SOM_C = 32
NCODE = SOM_R * SOM_C
BLK = 256
GRID = B // BLK
NSLOT = 5

_TROWS = NCODE + 8
_TW = 128
_BIG = (1 << 30)


def _tc1_body(x_ref, we_ref, be_ref, e_ref, et_ref,
              ze_ref, zd_ref, k_ref, g_ref, tab_ref):
    x = x_ref[...]
    E = e_ref[...]
    Et = et_ref[...]
    ze = jnp.dot(x, we_ref[...], preferred_element_type=jnp.float32) + be_ref[...]
    ze_ref[...] = ze

    enorm = jnp.sum(Et * Et, axis=0, keepdims=True)
    cross = jnp.dot(ze, Et, preferred_element_type=jnp.float32,
                    precision=lax.Precision.HIGHEST)
    c = enorm - 2.0 * cross
    znorm = jnp.sum(ze * ze, axis=1, keepdims=True)
    zd_ref[...] = znorm + c

    cmin = jnp.min(c, axis=1, keepdims=True)
    iota = lax.broadcasted_iota(jnp.int32, (BLK, NCODE), 1)
    k = jnp.min(jnp.where(c <= cmin, iota, _BIG), axis=1, keepdims=True)
    k_ref[...] = k

    k1 = k // SOM_C
    k2 = k % SOM_C
    row_iota = lax.broadcasted_iota(jnp.int32, (BLK, 1), 0)
    zpad = NCODE + (row_iota & 7)
    up = jnp.where(k1 < (SOM_R - 1), k + SOM_C, zpad)
    down = jnp.where(k1 > 0, k - SOM_C, zpad)
    left = jnp.where(k2 > 0, k - 1, zpad)
    g_ref[...] = jnp.concatenate([k, up, down, zpad, left], axis=1)

    @pl.when(pl.program_id(0) == 0)
    def _():
        tab_ref[...] = jnp.zeros((_TROWS, _TW), jnp.float32)
        tab_ref[pl.ds(0, NCODE), pl.ds(0, LATENT)] = E


def _tc1_call(x, W_enc, b_enc2, E_flat, E_t):
    full = lambda s: pl.BlockSpec(s, lambda i: (0,) * len(s))
    return pl.pallas_call(
        _tc1_body,
        grid=(GRID,),
        in_specs=[
            pl.BlockSpec((BLK, D_IN), lambda i: (i, 0)),
            full((D_IN, LATENT)),
            full((1, LATENT)),
            full((NCODE, LATENT)),
            full((LATENT, NCODE)),
        ],
        out_specs=[
            pl.BlockSpec((BLK, LATENT), lambda i: (i, 0)),
            pl.BlockSpec((BLK, NCODE), lambda i: (i, 0)),
            pl.BlockSpec((BLK, 1), lambda i: (i, 0)),
            pl.BlockSpec((BLK, NSLOT), lambda i: (i, 0)),
            full((_TROWS, _TW)),
        ],
        out_shape=[
            jax.ShapeDtypeStruct((B, LATENT), jnp.float32),
            jax.ShapeDtypeStruct((B, NCODE), jnp.float32),
            jax.ShapeDtypeStruct((B, 1), jnp.int32),
            jax.ShapeDtypeStruct((B, NSLOT), jnp.int32),
            jax.ShapeDtypeStruct((_TROWS, _TW), jnp.float32),
        ],
        compiler_params=pltpu.CompilerParams(
            dimension_semantics=("arbitrary",),
        ),
    )(x, W_enc, b_enc2, E_flat, E_t)


def _tc2_body(ze_ref, k_ref, e_ref, wq_ref, bq_ref, wde_ref, bde_ref,
              zq_ref, xq_ref, xe_ref):
    ze = ze_ref[...]
    k = k_ref[...]
    iota = lax.broadcasted_iota(jnp.int32, (BLK, NCODE), 1)
    onehot = (iota == k).astype(jnp.float32)
    zq = jnp.dot(onehot, e_ref[...], preferred_element_type=jnp.float32,
                 precision=lax.Precision.HIGHEST)
    zq_ref[...] = zq
    xq_ref[...] = jnp.dot(zq, wq_ref[...], preferred_element_type=jnp.float32) + bq_ref[...]
    xe_ref[...] = jnp.dot(ze, wde_ref[...], preferred_element_type=jnp.float32) + bde_ref[...]


def _tc2_call(z_e, k2d, E_flat, W_dec_q, b_dec_q2, W_dec_e, b_dec_e2):
    full = lambda s: pl.BlockSpec(s, lambda i: (0,) * len(s))
    return pl.pallas_call(
        _tc2_body,
        grid=(GRID,),
        in_specs=[
            pl.BlockSpec((BLK, LATENT), lambda i: (i, 0)),
            pl.BlockSpec((BLK, 1), lambda i: (i, 0)),
            full((NCODE, LATENT)),
            full((LATENT, D_IN)),
            full((1, D_IN)),
            full((LATENT, D_IN)),
            full((1, D_IN)),
        ],
        out_specs=[
            pl.BlockSpec((BLK, LATENT), lambda i: (i, 0)),
            pl.BlockSpec((BLK, D_IN), lambda i: (i, 0)),
            pl.BlockSpec((BLK, D_IN), lambda i: (i, 0)),
        ],
        out_shape=[
            jax.ShapeDtypeStruct((B, LATENT), jnp.float32),
            jax.ShapeDtypeStruct((B, D_IN), jnp.float32),
            jax.ShapeDtypeStruct((B, D_IN), jnp.float32),
        ],
        compiler_params=pltpu.CompilerParams(
            dimension_semantics=("arbitrary",),
        ),
    )(z_e, k2d, E_flat, W_dec_q, b_dec_q2, W_dec_e, b_dec_e2)


_NC = 2
_NS = 16
_NW = _NC * _NS
_NG = NSLOT * B
_BPW = _NG // _NW
_CH = _BPW // 2


@functools.lru_cache(maxsize=None)
def _sc_gather_fn():
    mesh = plsc.VectorSubcoreMesh(core_axis_name="c", subcore_axis_name="s")

    @functools.partial(
        pl.kernel,
        mesh=mesh,
        out_type=jax.ShapeDtypeStruct((_NG, _TW), jnp.float32),
        scratch_types=[
            pltpu.VMEM((_CH,), jnp.int32),
            pltpu.VMEM((_CH,), jnp.int32),
            pltpu.VMEM((_CH, _TW), jnp.float32),
            pltpu.VMEM((_CH, _TW), jnp.float32),
            pltpu.SemaphoreType.DMA,
            pltpu.SemaphoreType.DMA,
        ],
    )
    def _sc_gather(table_hbm, idx_hbm, out_hbm,
                   idx_a, idx_b, rows_a, rows_b, sem_a, sem_b):
        wid = lax.axis_index("s") * _NC + lax.axis_index("c")
        base = wid * _BPW
        pltpu.sync_copy(idx_hbm.at[pl.ds(base, _CH)], idx_a)
        pltpu.sync_copy(idx_hbm.at[pl.ds(base + _CH, _CH)], idx_b)
        ca = pltpu.async_copy(table_hbm.at[idx_a], rows_a, sem_a)
        cb = pltpu.async_copy(table_hbm.at[idx_b], rows_b, sem_b)
        ca.wait()
        pltpu.sync_copy(rows_a, out_hbm.at[pl.ds(base, _CH)])
        cb.wait()
        pltpu.sync_copy(rows_b, out_hbm.at[pl.ds(base + _CH, _CH)])

    return _sc_gather


def kernel(x, W_enc, b_enc, W_dec_q, b_dec_q, W_dec_e, b_dec_e, embeddings):
    E_flat = embeddings.reshape(NCODE, LATENT)
    z_e, zdist, k2d, gidx, table = _tc1_call(
        x, W_enc, b_enc.reshape(1, LATENT), E_flat, E_flat.T)
    k = k2d.reshape(B)

    nb = _sc_gather_fn()(table, gidx.reshape(_NG))
    z_q, x_q, x_e = _tc2_call(
        z_e, k2d, E_flat,
        W_dec_q, b_dec_q.reshape(1, D_IN),
        W_dec_e, b_dec_e.reshape(1, D_IN))

    z_q_neighbors = nb[:, :LATENT].reshape(B, NSLOT, LATENT)
    return (x_e, x_q, z_e, z_q, z_q_neighbors, k, zdist)


# speedup vs baseline: 1.9679x; 1.0410x over previous
import functools

import jax
import jax.numpy as jnp
from jax import lax
from jax.experimental import pallas as pl
from jax.experimental.pallas import tpu as pltpu
from jax.experimental.pallas import tpu_sc as plsc

B = 1024
D_IN = 512
LATENT = 64
SOM_R = 32
SOM_C = 32
NCODE = SOM_R * SOM_C
BLK = 256
GRID = B // BLK
NSLOT = 5

_TROWS = NCODE + 8
_TW = 128
_BIG = (1 << 30)


def _tc1_body(x_ref, we_ref, be_ref, e_ref,
              ze_ref, zd_ref, k_ref, g_ref, tab_ref):
    x = x_ref[...]
    E = e_ref[...].reshape(NCODE, LATENT)
    Et = E.T
    ze = jnp.dot(x, we_ref[...], preferred_element_type=jnp.float32) + be_ref[...]
    ze_ref[...] = ze

    enorm = jnp.sum(Et * Et, axis=0, keepdims=True)
    cross = jnp.dot(ze, Et, preferred_element_type=jnp.float32,
                    precision=lax.Precision.HIGHEST)
    c = enorm - 2.0 * cross
    znorm = jnp.sum(ze * ze, axis=1, keepdims=True)
    zd_ref[...] = znorm + c

    cmin = jnp.min(c, axis=1, keepdims=True)
    iota = lax.broadcasted_iota(jnp.int32, (BLK, NCODE), 1)
    k = jnp.min(jnp.where(c <= cmin, iota, _BIG), axis=1, keepdims=True)
    k_ref[...] = k

    k1 = k // SOM_C
    k2 = k % SOM_C
    row_iota = lax.broadcasted_iota(jnp.int32, (BLK, 1), 0)
    zpad = NCODE + (row_iota & 7)
    up = jnp.where(k1 < (SOM_R - 1), k + SOM_C, zpad)
    down = jnp.where(k1 > 0, k - SOM_C, zpad)
    left = jnp.where(k2 > 0, k - 1, zpad)
    g_ref[...] = jnp.concatenate([k, up, down, zpad, left], axis=1)

    @pl.when(pl.program_id(0) == 0)
    def _():
        tab_ref[...] = jnp.zeros((_TROWS, _TW), jnp.float32)
        tab_ref[pl.ds(0, NCODE), pl.ds(0, LATENT)] = E


def _tc1_call(x, W_enc, b_enc2, emb):
    full = lambda s: pl.BlockSpec(s, lambda i: (0,) * len(s))
    return pl.pallas_call(
        _tc1_body,
        grid=(GRID,),
        in_specs=[
            pl.BlockSpec((BLK, D_IN), lambda i: (i, 0)),
            full((D_IN, LATENT)),
            full((1, LATENT)),
            full((SOM_R, SOM_C, LATENT)),
        ],
        out_specs=[
            pl.BlockSpec((BLK, LATENT), lambda i: (i, 0)),
            pl.BlockSpec((BLK, NCODE), lambda i: (i, 0)),
            pl.BlockSpec((BLK, 1), lambda i: (i, 0)),
            pl.BlockSpec((BLK, NSLOT), lambda i: (i, 0)),
            full((_TROWS, _TW)),
        ],
        out_shape=[
            jax.ShapeDtypeStruct((B, LATENT), jnp.float32),
            jax.ShapeDtypeStruct((B, NCODE), jnp.float32),
            jax.ShapeDtypeStruct((B, 1), jnp.int32),
            jax.ShapeDtypeStruct((B, NSLOT), jnp.int32),
            jax.ShapeDtypeStruct((_TROWS, _TW), jnp.float32),
        ],
        compiler_params=pltpu.CompilerParams(
            dimension_semantics=("arbitrary",),
        ),
    )(x, W_enc, b_enc2, emb)


def _tc2_body(ze_ref, k_ref, e_ref, wq_ref, bq_ref, wde_ref, bde_ref,
              zq_ref, xq_ref, xe_ref):
    ze = ze_ref[...]
    k = k_ref[...]
    iota = lax.broadcasted_iota(jnp.int32, (BLK, NCODE), 1)
    onehot = (iota == k).astype(jnp.float32)
    zq = jnp.dot(onehot, e_ref[...].reshape(NCODE, LATENT),
                 preferred_element_type=jnp.float32,
                 precision=lax.Precision.HIGHEST)
    zq_ref[...] = zq
    xq_ref[...] = jnp.dot(zq, wq_ref[...], preferred_element_type=jnp.float32) + bq_ref[...]
    xe_ref[...] = jnp.dot(ze, wde_ref[...], preferred_element_type=jnp.float32) + bde_ref[...]


def _tc2_call(z_e, k2d, emb, W_dec_q, b_dec_q2, W_dec_e, b_dec_e2):
    full = lambda s: pl.BlockSpec(s, lambda i: (0,) * len(s))
    return pl.pallas_call(
        _tc2_body,
        grid=(GRID,),
        in_specs=[
            pl.BlockSpec((BLK, LATENT), lambda i: (i, 0)),
            pl.BlockSpec((BLK, 1), lambda i: (i, 0)),
            full((SOM_R, SOM_C, LATENT)),
            full((LATENT, D_IN)),
            full((1, D_IN)),
            full((LATENT, D_IN)),
            full((1, D_IN)),
        ],
        out_specs=[
            pl.BlockSpec((BLK, LATENT), lambda i: (i, 0)),
            pl.BlockSpec((BLK, D_IN), lambda i: (i, 0)),
            pl.BlockSpec((BLK, D_IN), lambda i: (i, 0)),
        ],
        out_shape=[
            jax.ShapeDtypeStruct((B, LATENT), jnp.float32),
            jax.ShapeDtypeStruct((B, D_IN), jnp.float32),
            jax.ShapeDtypeStruct((B, D_IN), jnp.float32),
        ],
        compiler_params=pltpu.CompilerParams(
            dimension_semantics=("arbitrary",),
        ),
    )(z_e, k2d, emb, W_dec_q, b_dec_q2, W_dec_e, b_dec_e2)


_NC = 2
_NS = 16
_NW = _NC * _NS
_NG = NSLOT * B
_BPW = _NG // _NW
_NCH = 4
_CH = _BPW // _NCH


@functools.lru_cache(maxsize=None)
def _sc_gather_fn():
    mesh = plsc.VectorSubcoreMesh(core_axis_name="c", subcore_axis_name="s")

    @functools.partial(
        pl.kernel,
        mesh=mesh,
        out_type=jax.ShapeDtypeStruct((_NG, _TW), jnp.float32),
        scratch_types=(
            [pltpu.VMEM((_CH,), jnp.int32)] * _NCH
            + [pltpu.VMEM((_CH, _TW), jnp.float32)] * _NCH
            + [pltpu.SemaphoreType.DMA] * 3
        ),
    )
    def _sc_gather(table_hbm, idx_hbm, out_hbm, *refs):
        idx_v = refs[:_NCH]
        rows_v = refs[_NCH:2 * _NCH]
        isem, gsem, wsem = refs[2 * _NCH:]
        wid = lax.axis_index("s") * _NC + lax.axis_index("c")
        base = wid * _BPW
        ic = [pltpu.async_copy(idx_hbm.at[pl.ds(base + j * _CH, _CH)],
                               idx_v[j], isem) for j in range(_NCH)]
        for c in ic:
            c.wait()
        gc = [pltpu.async_copy(table_hbm.at[idx_v[j]], rows_v[j], gsem)
              for j in range(_NCH)]
        for c in gc:
            c.wait()
        wc = [pltpu.async_copy(rows_v[j],
                               out_hbm.at[pl.ds(base + j * _CH, _CH)], wsem)
              for j in range(_NCH)]
        for c in wc:
            c.wait()

    return _sc_gather


def kernel(x, W_enc, b_enc, W_dec_q, b_dec_q, W_dec_e, b_dec_e, embeddings):
    z_e, zdist, k2d, gidx, table = _tc1_call(
        x, W_enc, b_enc.reshape(1, LATENT), embeddings)
    k = k2d.reshape(B)

    nb = _sc_gather_fn()(table, gidx.reshape(_NG))
    z_q, x_q, x_e = _tc2_call(
        z_e, k2d, embeddings,
        W_dec_q, b_dec_q.reshape(1, D_IN),
        W_dec_e, b_dec_e.reshape(1, D_IN))

    z_q_neighbors = nb[:, :LATENT].reshape(B, NSLOT, LATENT)
    return (x_e, x_q, z_e, z_q, z_q_neighbors, k, zdist)


# speedup vs baseline: 2.1636x; 1.0994x over previous
import functools

import jax
import jax.numpy as jnp
from jax import lax
from jax.experimental import pallas as pl
from jax.experimental.pallas import tpu as pltpu
from jax.experimental.pallas import tpu_sc as plsc

B = 1024
D_IN = 512
LATENT = 64
SOM_R = 32
SOM_C = 32
NCODE = SOM_R * SOM_C
BLK = 256
GRID = B // BLK
NSLOT = 5

_TROWS = NCODE + 8
_TW = 128
_BIG = (1 << 30)


def _tc1_body(x_ref, wet_ref, be_ref, e_ref,
              ze_ref, zet_ref, zd_ref, k_ref, g_ref, tab_ref):
    x = x_ref[...]
    E = e_ref[...].reshape(NCODE, LATENT)
    Et = E.T
    We = wet_ref[...].T
    ze = jnp.dot(x, We, preferred_element_type=jnp.float32) + be_ref[...]
    ze_ref[...] = ze
    zet_ref[...] = ze.T

    enorm = jnp.sum(Et * Et, axis=0, keepdims=True)
    cross = jnp.dot(ze, Et, preferred_element_type=jnp.float32,
                    precision=lax.Precision.HIGHEST)
    c = enorm - 2.0 * cross
    znorm = jnp.sum(ze * ze, axis=1, keepdims=True)
    zd_ref[...] = znorm + c

    cmin = jnp.min(c, axis=1, keepdims=True)
    iota = lax.broadcasted_iota(jnp.int32, (BLK, NCODE), 1)
    k = jnp.min(jnp.where(c <= cmin, iota, _BIG), axis=1, keepdims=True)
    k_ref[...] = k

    k1 = k // SOM_C
    k2 = k % SOM_C
    row_iota = lax.broadcasted_iota(jnp.int32, (BLK, 1), 0)
    zpad = NCODE + (row_iota & 7)
    up = jnp.where(k1 < (SOM_R - 1), k + SOM_C, zpad)
    down = jnp.where(k1 > 0, k - SOM_C, zpad)
    left = jnp.where(k2 > 0, k - 1, zpad)
    g_ref[...] = jnp.concatenate([k, up, down, zpad, left], axis=1)

    @pl.when(pl.program_id(0) == 0)
    def _():
        tab_ref[...] = jnp.zeros((_TROWS, _TW), jnp.float32)
        tab_ref[pl.ds(0, NCODE), pl.ds(0, LATENT)] = E


def _tc1_call(x, W_enc_t, b_enc2, emb):
    full = lambda s: pl.BlockSpec(s, lambda i: (0,) * len(s))
    return pl.pallas_call(
        _tc1_body,
        grid=(GRID,),
        in_specs=[
            pl.BlockSpec((BLK, D_IN), lambda i: (i, 0)),
            full((LATENT, D_IN)),
            full((1, LATENT)),
            full((SOM_R, SOM_C, LATENT)),
        ],
        out_specs=[
            pl.BlockSpec((BLK, LATENT), lambda i: (i, 0)),
            pl.BlockSpec((LATENT, BLK), lambda i: (0, i)),
            pl.BlockSpec((BLK, NCODE), lambda i: (i, 0)),
            pl.BlockSpec((BLK, 1), lambda i: (i, 0)),
            pl.BlockSpec((BLK, NSLOT), lambda i: (i, 0)),
            full((_TROWS, _TW)),
        ],
        out_shape=[
            jax.ShapeDtypeStruct((B, LATENT), jnp.float32),
            jax.ShapeDtypeStruct((LATENT, B), jnp.float32),
            jax.ShapeDtypeStruct((B, NCODE), jnp.float32),
            jax.ShapeDtypeStruct((B, 1), jnp.int32),
            jax.ShapeDtypeStruct((B, NSLOT), jnp.int32),
            jax.ShapeDtypeStruct((_TROWS, _TW), jnp.float32),
        ],
        compiler_params=pltpu.CompilerParams(
            dimension_semantics=("arbitrary",),
        ),
    )(x, W_enc_t, b_enc2, emb)


def _tc2_body(ze_ref, k_ref, e_ref, wq_ref, bq_ref, wde_ref, bde_ref,
              zq_ref, xq_ref, xe_ref):
    ze = ze_ref[...]
    k = k_ref[...]
    iota = lax.broadcasted_iota(jnp.int32, (BLK, NCODE), 1)
    onehot = (iota == k).astype(jnp.float32)
    zq = jnp.dot(onehot, e_ref[...].reshape(NCODE, LATENT),
                 preferred_element_type=jnp.float32,
                 precision=lax.Precision.HIGHEST)
    zq_ref[...] = zq.T
    xq_ref[...] = jnp.dot(zq, wq_ref[...], preferred_element_type=jnp.float32) + bq_ref[...]
    xe_ref[...] = jnp.dot(ze, wde_ref[...], preferred_element_type=jnp.float32) + bde_ref[...]


def _tc2_call(z_e, k2d, emb, W_dec_q, b_dec_q2, W_dec_e, b_dec_e2):
    full = lambda s: pl.BlockSpec(s, lambda i: (0,) * len(s))
    return pl.pallas_call(
        _tc2_body,
        grid=(GRID,),
        in_specs=[
            pl.BlockSpec((BLK, LATENT), lambda i: (i, 0)),
            pl.BlockSpec((BLK, 1), lambda i: (i, 0)),
            full((SOM_R, SOM_C, LATENT)),
            full((LATENT, D_IN)),
            full((1, D_IN)),
            full((LATENT, D_IN)),
            full((1, D_IN)),
        ],
        out_specs=[
            pl.BlockSpec((LATENT, BLK), lambda i: (0, i)),
            pl.BlockSpec((BLK, D_IN), lambda i: (i, 0)),
            pl.BlockSpec((BLK, D_IN), lambda i: (i, 0)),
        ],
        out_shape=[
            jax.ShapeDtypeStruct((LATENT, B), jnp.float32),
            jax.ShapeDtypeStruct((B, D_IN), jnp.float32),
            jax.ShapeDtypeStruct((B, D_IN), jnp.float32),
        ],
        compiler_params=pltpu.CompilerParams(
            dimension_semantics=("arbitrary",),
        ),
    )(z_e, k2d, emb, W_dec_q, b_dec_q2, W_dec_e, b_dec_e2)


_NC = 2
_NS = 16
_NW = _NC * _NS
_NG = NSLOT * B
_BPW = _NG // _NW
_NCH = 4
_CH = _BPW // _NCH


@functools.lru_cache(maxsize=None)
def _sc_gather_fn():
    mesh = plsc.VectorSubcoreMesh(core_axis_name="c", subcore_axis_name="s")

    @functools.partial(
        pl.kernel,
        mesh=mesh,
        out_type=jax.ShapeDtypeStruct((_NG, _TW), jnp.float32),
        scratch_types=(
            [pltpu.VMEM((_CH,), jnp.int32)] * _NCH
            + [pltpu.VMEM((_CH, _TW), jnp.float32)] * _NCH
            + [pltpu.SemaphoreType.DMA] * 3
        ),
    )
    def _sc_gather(table_hbm, idx_hbm, out_hbm, *refs):
        idx_v = refs[:_NCH]
        rows_v = refs[_NCH:2 * _NCH]
        isem, gsem, wsem = refs[2 * _NCH:]
        wid = lax.axis_index("s") * _NC + lax.axis_index("c")
        base = wid * _BPW
        ic = [pltpu.async_copy(idx_hbm.at[pl.ds(base + j * _CH, _CH)],
                               idx_v[j], isem) for j in range(_NCH)]
        for c in ic:
            c.wait()
        gc = [pltpu.async_copy(table_hbm.at[idx_v[j]], rows_v[j], gsem)
              for j in range(_NCH)]
        for c in gc:
            c.wait()
        wc = [pltpu.async_copy(rows_v[j],
                               out_hbm.at[pl.ds(base + j * _CH, _CH)], wsem)
              for j in range(_NCH)]
        for c in wc:
            c.wait()

    return _sc_gather


def kernel(x, W_enc, b_enc, W_dec_q, b_dec_q, W_dec_e, b_dec_e, embeddings):
    ze_i, ze_t, zdist, k2d, gidx, table = _tc1_call(
        x, W_enc.T, b_enc.reshape(1, LATENT), embeddings)
    k = k2d.reshape(B)
    z_e = ze_t.T

    nb = _sc_gather_fn()(table, gidx.reshape(_NG))
    zq_t, x_q, x_e = _tc2_call(
        ze_i, k2d, embeddings,
        W_dec_q, b_dec_q.reshape(1, D_IN),
        W_dec_e, b_dec_e.reshape(1, D_IN))
    z_q = zq_t.T

    z_q_neighbors = nb[:, :LATENT].reshape(B, NSLOT, LATENT)
    return (x_e, x_q, z_e, z_q, z_q_neighbors, k, zdist)
